# Initial kernel scaffold; baseline (speedup 1.0000x reference)
import jax
import jax.numpy as jnp
from jax.experimental import pallas as pl


def kernel(bilateral_grid, guidemap):
    raise NotImplementedError("write your pallas kernel here")


# speedup vs baseline: 132.4498x; 132.4498x over previous
import functools

import jax
import jax.numpy as jnp
from jax import lax
from jax.experimental import pallas as pl
from jax.experimental.pallas import tpu as pltpu
from jax.experimental.pallas import tpu_sc as plsc

N, C, D, GH, GW = 4, 12, 8, 16, 16
H = W = 512
NWORK = 32
ROWS_PER_W = (N * H) // NWORK
CHUNK = 8
RT = D * GH * C
TBL = GW * RT
NGRP = W // 16


def _sc_body(t2_hbm, guide_hbm, idx0_hbm, idx1_hbm, frac_hbm, out_hbm,
             t2_v, rt_v, gd_v, out_v, idx0_v, idx1_v, frac_v):
    wid = lax.axis_index("c") * 16 + lax.axis_index("s")
    n = wid // 8
    iblk = (wid % 8) * ROWS_PER_W

    pltpu.sync_copy(t2_hbm.at[n], t2_v)
    pltpu.sync_copy(idx0_hbm, idx0_v)
    pltpu.sync_copy(idx1_hbm, idx1_v)
    pltpu.sync_copy(frac_hbm, frac_v)

    def chunk_body(ch, carry):
        i0 = iblk + ch * CHUNK
        pltpu.sync_copy(guide_hbm.at[n, pl.ds(i0, CHUNK), :], gd_v)

        def row_body(r, carry):
            i = i0 + r
            i15 = i * (GW - 1)
            x0i = i15 // (H - 1)
            wx = i15.astype(jnp.float32) * (1.0 / (H - 1)) - x0i.astype(jnp.float32)
            x1i = jnp.minimum(x0i + 1, GW - 1)
            x0 = x0i * RT
            x1 = x1i * RT

            def rt_body(k, carry):
                off = k * 16
                v0 = t2_v[pl.ds(x0 + off, 16)]
                v1 = t2_v[pl.ds(x1 + off, 16)]
                rt_v[pl.ds(off, 16)] = v0 + wx * (v1 - v0)
                return carry

            lax.fori_loop(0, RT // 16, rt_body, 0, unroll=4)

            def grp_body(gj, carry):
                j0 = gj * 16
                g = gd_v[r, pl.ds(j0, 16)]
                iz = jnp.clip((g + 1.0) * (0.5 * (D - 1)), 0.0, float(D - 1))
                z0 = iz.astype(jnp.int32)
                wz = iz - z0.astype(jnp.float32)
                z1 = jnp.minimum(z0 + 1, D - 1)
                y0o = idx0_v[pl.ds(j0, 16)] * C
                y1o = idx1_v[pl.ds(j0, 16)] * C
                wy = frac_v[pl.ds(j0, 16)]
                b00 = z0 * (GH * C) + y0o
                b01 = z0 * (GH * C) + y1o
                b10 = z1 * (GH * C) + y0o
                b11 = z1 * (GH * C) + y1o
                omz = 1.0 - wz
                omy = 1.0 - wy
                w00 = omz * omy
                w01 = omz * wy
                w10 = wz * omy
                w11 = wz * wy
                for c in range(C):
                    v = (plsc.load_gather(rt_v, [b00 + c]) * w00 +
                         plsc.load_gather(rt_v, [b01 + c]) * w01 +
                         plsc.load_gather(rt_v, [b10 + c]) * w10 +
                         plsc.load_gather(rt_v, [b11 + c]) * w11)
                    out_v[c, r, pl.ds(j0, 16)] = v
                return carry

            lax.fori_loop(0, NGRP, grp_body, 0)
            return carry

        lax.fori_loop(0, CHUNK, row_body, 0)
        pltpu.sync_copy(out_v, out_hbm.at[n, :, pl.ds(i0, CHUNK), :])
        return carry

    lax.fori_loop(0, ROWS_PER_W // CHUNK, chunk_body, 0)


@jax.jit
def kernel(bilateral_grid, guidemap):
    t2 = jnp.transpose(bilateral_grid, (0, 4, 2, 3, 1)).reshape(N, TBL)

    t = (jnp.arange(512, dtype=jnp.float32) / (H - 1)) * 2.0 - 1.0
    pos = jnp.clip((t + 1.0) * 0.5 * (GW - 1), 0.0, float(GW - 1))
    f0 = jnp.floor(pos)
    idx0 = f0.astype(jnp.int32)
    idx1 = jnp.minimum(idx0 + 1, GW - 1)
    frac = pos - f0

    mesh = plsc.VectorSubcoreMesh(core_axis_name="c", subcore_axis_name="s")
    run = functools.partial(
        pl.kernel,
        mesh=mesh,
        compiler_params=pltpu.CompilerParams(needs_layout_passes=False),
        out_type=jax.ShapeDtypeStruct((N, C, H, W), jnp.float32),
        scratch_types=[
            pltpu.VMEM((TBL,), jnp.float32),
            pltpu.VMEM((RT,), jnp.float32),
            pltpu.VMEM((CHUNK, W), jnp.float32),
            pltpu.VMEM((C, CHUNK, W), jnp.float32),
            pltpu.VMEM((512,), jnp.int32),
            pltpu.VMEM((512,), jnp.int32),
            pltpu.VMEM((512,), jnp.float32),
        ],
    )(_sc_body)
    return run(t2, guidemap, idx0, idx1, frac)


# speedup vs baseline: 174.3682x; 1.3165x over previous
import functools

import jax
import jax.numpy as jnp
from jax import lax
from jax.experimental import pallas as pl
from jax.experimental.pallas import tpu as pltpu
from jax.experimental.pallas import tpu_sc as plsc

N, C, D, GH, GW = 4, 12, 8, 16, 16
H = W = 512
NWORK = 32
ROWS_PER_W = (N * H) // NWORK
CHUNK = 8
RT = D * GH * C
TBL = GW * RT
NGRP = W // 16


def _sc_body(t2_hbm, guide_hbm, idx0_hbm, idx1_hbm, frac_hbm, out_hbm,
             t2_v, rt_v, gd_v, out_v, idx0_v, idx1_v, frac_v):
    wid = lax.axis_index("c") * 16 + lax.axis_index("s")
    n = wid // 8
    iblk = (wid % 8) * ROWS_PER_W

    pltpu.sync_copy(t2_hbm.at[n], t2_v)
    pltpu.sync_copy(idx0_hbm, idx0_v)
    pltpu.sync_copy(idx1_hbm, idx1_v)
    pltpu.sync_copy(frac_hbm, frac_v)

    def chunk_body(ch, carry):
        i0 = iblk + ch * CHUNK
        pltpu.sync_copy(guide_hbm.at[n, pl.ds(i0, CHUNK), :], gd_v)

        def row_body(r, carry):
            i = i0 + r
            i15 = i * (GW - 1)
            x0i = i15 // (H - 1)
            wx = i15.astype(jnp.float32) * (1.0 / (H - 1)) - x0i.astype(jnp.float32)
            x1i = jnp.minimum(x0i + 1, GW - 1)
            x0 = x0i * RT
            x1 = x1i * RT

            def rt_body(k, carry):
                off = k * 16
                v0 = t2_v[pl.ds(x0 + off, 16)]
                v1 = t2_v[pl.ds(x1 + off, 16)]
                rt_v[pl.ds(off, 16)] = v0 + wx * (v1 - v0)
                return carry

            lax.fori_loop(0, RT // 16, rt_body, 0, unroll=4)

            def grp_body(gj, carry):
                j0 = gj * 16
                g = gd_v[r, pl.ds(j0, 16)]
                iz = jnp.clip((g + 1.0) * (0.5 * (D - 1)), 0.0, float(D - 1))
                z0 = iz.astype(jnp.int32)
                wz = iz - z0.astype(jnp.float32)
                z1 = jnp.minimum(z0 + 1, D - 1)
                y0o = idx0_v[pl.ds(j0, 16)] * C
                y1o = idx1_v[pl.ds(j0, 16)] * C
                wy = frac_v[pl.ds(j0, 16)]
                b00 = z0 * (GH * C) + y0o
                b01 = z0 * (GH * C) + y1o
                b10 = z1 * (GH * C) + y0o
                b11 = z1 * (GH * C) + y1o
                omz = 1.0 - wz
                omy = 1.0 - wy
                w00 = omz * omy
                w01 = omz * wy
                w10 = wz * omy
                w11 = wz * wy
                g = [[plsc.load_gather(rt_v, [b + c])
                      for b in (b00, b01, b10, b11)]
                     for c in range(C)]
                for c in range(C):
                    g0, g1, g2, g3 = g[c]
                    v = (g0 * w00 + g1 * w01) + (g2 * w10 + g3 * w11)
                    out_v[c, r, pl.ds(j0, 16)] = v
                return carry

            lax.fori_loop(0, NGRP, grp_body, 0, unroll=2)
            return carry

        lax.fori_loop(0, CHUNK, row_body, 0)
        pltpu.sync_copy(out_v, out_hbm.at[n, :, pl.ds(i0, CHUNK), :])
        return carry

    lax.fori_loop(0, ROWS_PER_W // CHUNK, chunk_body, 0)


@jax.jit
def kernel(bilateral_grid, guidemap):
    t2 = jnp.transpose(bilateral_grid, (0, 4, 2, 3, 1)).reshape(N, TBL)

    t = (jnp.arange(512, dtype=jnp.float32) / (H - 1)) * 2.0 - 1.0
    pos = jnp.clip((t + 1.0) * 0.5 * (GW - 1), 0.0, float(GW - 1))
    f0 = jnp.floor(pos)
    idx0 = f0.astype(jnp.int32)
    idx1 = jnp.minimum(idx0 + 1, GW - 1)
    frac = pos - f0

    mesh = plsc.VectorSubcoreMesh(core_axis_name="c", subcore_axis_name="s")
    run = functools.partial(
        pl.kernel,
        mesh=mesh,
        compiler_params=pltpu.CompilerParams(needs_layout_passes=False),
        out_type=jax.ShapeDtypeStruct((N, C, H, W), jnp.float32),
        scratch_types=[
            pltpu.VMEM((TBL,), jnp.float32),
            pltpu.VMEM((RT,), jnp.float32),
            pltpu.VMEM((CHUNK, W), jnp.float32),
            pltpu.VMEM((C, CHUNK, W), jnp.float32),
            pltpu.VMEM((512,), jnp.int32),
            pltpu.VMEM((512,), jnp.int32),
            pltpu.VMEM((512,), jnp.float32),
        ],
    )(_sc_body)
    return run(t2, guidemap, idx0, idx1, frac)


# speedup vs baseline: 554.5877x; 3.1806x over previous
import functools

import jax
import jax.numpy as jnp
from jax import lax
from jax.experimental import pallas as pl
from jax.experimental.pallas import tpu as pltpu
from jax.experimental.pallas import tpu_sc as plsc

N, C, D, GH, GW = 4, 12, 8, 16, 16
H = W = 512
NWORK = 32
ROWS_PER_W = (N * H) // NWORK
CHUNK = 8
CP = C + 1
ZS = CP
YS = D * CP
RT = GH * YS
TBL = GW * RT
NGRP = W // 16


def _sc_body(t2_hbm, guide_hbm, idx0_hbm, idx1_hbm, frac_hbm, out_hbm,
             t2_v, rt_v, gd_v, out_v, idx0_v, idx1_v, frac_v):
    wid = lax.axis_index("c") * 16 + lax.axis_index("s")
    n = wid // 8
    iblk = (wid % 8) * ROWS_PER_W

    pltpu.sync_copy(t2_hbm.at[n], t2_v)
    pltpu.sync_copy(idx0_hbm, idx0_v)
    pltpu.sync_copy(idx1_hbm, idx1_v)
    pltpu.sync_copy(frac_hbm, frac_v)

    def chunk_body(ch, carry):
        i0 = iblk + ch * CHUNK
        pltpu.sync_copy(guide_hbm.at[n, pl.ds(i0, CHUNK), :], gd_v)

        def row_body(r, carry):
            i = i0 + r
            i15 = i * (GW - 1)
            x0i = i15 // (H - 1)
            wx = i15.astype(jnp.float32) * (1.0 / (H - 1)) - x0i.astype(jnp.float32)
            x1i = jnp.minimum(x0i + 1, GW - 1)
            x0 = x0i * RT
            x1 = x1i * RT

            def rt_body(k, carry):
                off = k * 16
                v0 = t2_v[pl.ds(x0 + off, 16)]
                v1 = t2_v[pl.ds(x1 + off, 16)]
                rt_v[pl.ds(off, 16)] = v0 + wx * (v1 - v0)
                return carry

            lax.fori_loop(0, RT // 16, rt_body, 0, unroll=4)

            def grp_body(gj, carry):
                j0 = gj * 16
                g = gd_v[r, pl.ds(j0, 16)]
                iz = jnp.clip((g + 1.0) * (0.5 * (D - 1)), 0.0, float(D - 1))
                z0 = iz.astype(jnp.int32)
                wz = iz - z0.astype(jnp.float32)
                z1 = jnp.minimum(z0 + 1, D - 1)
                y0o = idx0_v[pl.ds(j0, 16)] * YS
                y1o = idx1_v[pl.ds(j0, 16)] * YS
                wy = frac_v[pl.ds(j0, 16)]
                z0o = z0 * ZS
                z1o = z1 * ZS
                b00 = z0o + y0o
                b01 = z0o + y1o
                b10 = z1o + y0o
                b11 = z1o + y1o
                omz = 1.0 - wz
                omy = 1.0 - wy
                w00 = omz * omy
                w01 = omz * wy
                w10 = wz * omy
                w11 = wz * wy
                g = [[plsc.load_gather(rt_v, [b + c])
                      for b in (b00, b01, b10, b11)]
                     for c in range(C)]
                for c in range(C):
                    g0, g1, g2, g3 = g[c]
                    v = (g0 * w00 + g1 * w01) + (g2 * w10 + g3 * w11)
                    out_v[c, r, pl.ds(j0, 16)] = v
                return carry

            lax.fori_loop(0, NGRP, grp_body, 0, unroll=2)
            return carry

        lax.fori_loop(0, CHUNK, row_body, 0)
        pltpu.sync_copy(out_v, out_hbm.at[n, :, pl.ds(i0, CHUNK), :])
        return carry

    lax.fori_loop(0, ROWS_PER_W // CHUNK, chunk_body, 0)


@jax.jit
def kernel(bilateral_grid, guidemap):
    t2 = jnp.transpose(bilateral_grid, (0, 4, 3, 2, 1))
    t2 = jnp.pad(t2, ((0, 0), (0, 0), (0, 0), (0, 0), (0, CP - C)))
    t2 = t2.reshape(N, TBL)

    t = (jnp.arange(512, dtype=jnp.float32) / (H - 1)) * 2.0 - 1.0
    pos = jnp.clip((t + 1.0) * 0.5 * (GW - 1), 0.0, float(GW - 1))
    f0 = jnp.floor(pos)
    idx0 = f0.astype(jnp.int32)
    idx1 = jnp.minimum(idx0 + 1, GW - 1)
    frac = pos - f0

    mesh = plsc.VectorSubcoreMesh(core_axis_name="c", subcore_axis_name="s")
    run = functools.partial(
        pl.kernel,
        mesh=mesh,
        compiler_params=pltpu.CompilerParams(needs_layout_passes=False),
        out_type=jax.ShapeDtypeStruct((N, C, H, W), jnp.float32),
        scratch_types=[
            pltpu.VMEM((TBL,), jnp.float32),
            pltpu.VMEM((RT,), jnp.float32),
            pltpu.VMEM((CHUNK, W), jnp.float32),
            pltpu.VMEM((C, CHUNK, W), jnp.float32),
            pltpu.VMEM((512,), jnp.int32),
            pltpu.VMEM((512,), jnp.int32),
            pltpu.VMEM((512,), jnp.float32),
        ],
    )(_sc_body)
    return run(t2, guidemap, idx0, idx1, frac)
